# Initial kernel scaffold; baseline (speedup 1.0000x reference)
import jax
import jax.numpy as jnp
from jax.experimental import pallas as pl


def kernel(x, edge_index, edge_attr, batch, subgraph_batch, num_subgraphs, subgraph_id_batch, W_enc, b_enc, W_edge, b_edge, W_root, W_agg, b_gnn, gamma, beta, W1, b1, W2, b2):
    raise NotImplementedError("write your pallas kernel here")


# speedup vs baseline: 4.9461x; 4.9461x over previous
import functools

import jax
import jax.numpy as jnp
from jax import lax
from jax.experimental import pallas as pl
from jax.experimental.pallas import tpu as pltpu
from jax.experimental.pallas import tpu_sc as plsc

N = 10000
E = 320000
D = 128
DE = 16
G = 64
S = 512
EPS = 1e-5

NC = 2
NS = 16
NW = NC * NS
EPW = E // NW
CW = 80
NCHUNK = EPW // CW
RPT = 624
REXT_START = NS * RPT
REXT = N - REXT_START
RZB = 104

@functools.cache
def _make_sc_spmm():
    mesh = plsc.VectorSubcoreMesh(
        core_axis_name="c", subcore_axis_name="s", num_cores=NC, num_subcores=NS
    )
    return functools.partial(
        pl.kernel,
        out_type=jax.ShapeDtypeStruct((NC, N, D), jnp.float32),
        mesh=mesh,
        scratch_types=[
            pltpu.VMEM((NCHUNK, CW), jnp.int32),
            pltpu.VMEM((NCHUNK, CW), jnp.int32),
            pltpu.VMEM((CW, D), jnp.float32),
            pltpu.VMEM_SHARED((N, D), jnp.float32),
            pltpu.SemaphoreType.DMA,
        ],
    )(_sc_spmm_body)


def _sc_spmm_body(ei_hbm, h_hbm, out_hbm, src_v, dst_v, rows, acc, sem):
    cid = lax.axis_index("c")
    sid = lax.axis_index("s")
    wid = cid * NS + sid
    pltpu.sync_copy(ei_hbm.at[0, wid], src_v)
    pltpu.sync_copy(ei_hbm.at[1, wid], dst_v)

    @pl.loop(0, CW)
    def _zero_fill(i):
        for j in range(D // 16):
            rows[i, pl.ds(j * 16, 16)] = jnp.zeros((16,), jnp.float32)

    rs = pl.multiple_of(sid * RPT, 8)
    for m in range(RPT // CW):
        pltpu.sync_copy(rows, acc.at[pl.ds(rs + m * CW, CW)])
    pltpu.sync_copy(
        rows.at[pl.ds(0, RPT % CW)],
        acc.at[pl.ds(rs + (RPT // CW) * CW, RPT % CW)],
    )

    @pl.when(sid == NS - 1)
    def _zero_tail():
        pltpu.sync_copy(rows.at[pl.ds(0, REXT)], acc.at[pl.ds(REXT_START, REXT)])

    plsc.subcore_barrier()

    @pl.loop(0, NCHUNK)
    def _chunk(k):
        pltpu.async_copy(h_hbm.at[src_v.at[k]], rows, sem).wait()
        pltpu.sync_copy(rows, acc.at[dst_v.at[k]], add=True)

    plsc.subcore_barrier()
    pltpu.sync_copy(acc.at[pl.ds(rs, RPT)], out_hbm.at[cid, pl.ds(rs, RPT)])

    @pl.when(sid == NS - 1)
    def _write_tail():
        pltpu.sync_copy(
            acc.at[pl.ds(REXT_START, REXT)],
            out_hbm.at[cid, pl.ds(REXT_START, REXT)],
        )


@functools.cache
def _make_sc_pre():
    mesh = plsc.VectorSubcoreMesh(
        core_axis_name="c", subcore_axis_name="s", num_cores=NC, num_subcores=NS
    )
    return functools.partial(
        pl.kernel,
        out_type=jax.ShapeDtypeStruct((NC, N, D), jnp.float32),
        mesh=mesh,
        scratch_types=[
            pltpu.VMEM((NCHUNK, CW), jnp.int32),
            pltpu.VMEM((CW, D), jnp.float32),
            pltpu.VMEM((CW, DE), jnp.float32),
            pltpu.VMEM_SHARED((N, D), jnp.float32),
            pltpu.SemaphoreType.DMA,
        ],
    )(_sc_pre_body)


def _sc_pre_body(ei_hbm, ea_hbm, out_hbm, dst_v, ud_v, ea_v, acc, sem):
    cid = lax.axis_index("c")
    sid = lax.axis_index("s")
    wid = cid * NS + sid
    pltpu.sync_copy(ei_hbm.at[1, wid], dst_v)

    @pl.loop(0, CW)
    def _zero_fill(i):
        for j in range(D // 16):
            ud_v[i, pl.ds(j * 16, 16)] = jnp.zeros((16,), jnp.float32)

    rs = pl.multiple_of(sid * RPT, 8)
    for m in range(RPT // CW):
        pltpu.sync_copy(ud_v, acc.at[pl.ds(rs + m * CW, CW)])
    pltpu.sync_copy(
        ud_v.at[pl.ds(0, RPT % CW)],
        acc.at[pl.ds(rs + (RPT // CW) * CW, RPT % CW)],
    )

    @pl.when(sid == NS - 1)
    def _zero_tail():
        pltpu.sync_copy(ud_v.at[pl.ds(0, REXT)], acc.at[pl.ds(REXT_START, REXT)])

    plsc.subcore_barrier()

    @pl.loop(0, CW)
    def _ones_fill(i):
        ud_v[i, pl.ds(DE, 16)] = jnp.ones((16,), jnp.float32)

    ebase = pl.multiple_of(wid * EPW, 8)

    @pl.loop(0, NCHUNK)
    def _chunk(k):
        pltpu.async_copy(
            ea_hbm.at[pl.ds(pl.multiple_of(ebase + k * CW, 8), CW)], ea_v, sem
        ).wait()

        @pl.loop(0, CW)
        def _fill(i):
            ud_v[i, pl.ds(0, DE)] = ea_v[i, :]

        pltpu.sync_copy(ud_v, acc.at[dst_v.at[k]], add=True)

    plsc.subcore_barrier()
    pltpu.sync_copy(acc.at[pl.ds(rs, RPT)], out_hbm.at[cid, pl.ds(rs, RPT)])

    @pl.when(sid == NS - 1)
    def _write_tail():
        pltpu.sync_copy(
            acc.at[pl.ds(REXT_START, REXT)], out_hbm.at[cid, pl.ds(REXT_START, REXT)]
        )


def _enc_body(x_ref, w_ref, b_ref, o_ref):
    o_ref[...] = (
        jnp.dot(x_ref[...], w_ref[...], preferred_element_type=jnp.float32)
        + b_ref[...]
    )


def _tc_encode(x, W_enc, b_enc):
    return pl.pallas_call(
        _enc_body, out_shape=jax.ShapeDtypeStruct((N, D), jnp.float32)
    )(x, W_enc, b_enc)


def _layer_body(h_ref, p_ref, pre_ref, we_ref, be_ref, wr_ref, wa_ref,
                bg_ref, ga_ref, bt_ref, o_ref):
    pre = pre_ref[0] + pre_ref[1]
    ea = pre[:, 0:DE]
    deg = pre[:, DE:DE + 1]
    agg = (
        p_ref[0]
        + p_ref[1]
        + jnp.dot(ea, we_ref[...], preferred_element_type=jnp.float32)
        + deg * be_ref[...]
    )
    z = (
        jnp.dot(h_ref[...], wr_ref[...], preferred_element_type=jnp.float32)
        + jnp.dot(agg, wa_ref[...], preferred_element_type=jnp.float32)
        + bg_ref[...]
    )
    mu = jnp.mean(z, axis=0, keepdims=True)
    zc = z - mu
    var = jnp.mean(zc * zc, axis=0, keepdims=True)
    zn = zc * lax.rsqrt(var + EPS) * ga_ref[...] + bt_ref[...]
    o_ref[...] = jnp.maximum(zn, 0.0)


def _tc_layer(h, P, pre_p, We, be, Wr, Wa, bg, ga, bt):
    return pl.pallas_call(
        _layer_body, out_shape=jax.ShapeDtypeStruct((N, D), jnp.float32)
    )(h, P, pre_p, We, be, Wr, Wa, bg, ga, bt)


def _pool_body(h_ref, b_ref, sb_ref, ns_ref, gi_ref, w1_ref, b1_ref, w2_ref,
               b2_ref, o_ref):
    ii = lax.broadcasted_iota(jnp.int32, (G, G), 0)
    jj = lax.broadcasted_iota(jnp.int32, (G, G), 1)
    tri = (jj < ii).astype(jnp.float32)
    offs_g = jnp.dot(tri, ns_ref[...].astype(jnp.float32),
                     preferred_element_type=jnp.float32)
    b_oh = (b_ref[...] == lax.broadcasted_iota(jnp.int32, (N, G), 1)).astype(
        jnp.float32
    )
    offs_n = jnp.dot(b_oh, offs_g, preferred_element_type=jnp.float32)
    sid = sb_ref[...].astype(jnp.float32) + offs_n
    oh = (
        sid == lax.broadcasted_iota(jnp.int32, (N, S), 1).astype(jnp.float32)
    ).astype(jnp.float32)
    dn = (((0,), (0,)), ((), ()))
    sums = lax.dot_general(oh, h_ref[...], dn,
                           preferred_element_type=jnp.float32)
    ones_n = jnp.zeros((N, 1), jnp.float32) + 1.0
    cnts = lax.dot_general(oh, ones_n, dn,
                           preferred_element_type=jnp.float32)
    hs = sums / jnp.maximum(cnts, 1.0)
    oh2 = (gi_ref[...] == lax.broadcasted_iota(jnp.int32, (S, G), 1)).astype(
        jnp.float32
    )
    gsum = lax.dot_general(oh2, hs, dn, preferred_element_type=jnp.float32)
    ones_s = jnp.zeros((S, 1), jnp.float32) + 1.0
    gcnt = lax.dot_general(oh2, ones_s, dn, preferred_element_type=jnp.float32)
    hg = gsum / jnp.maximum(gcnt, 1.0)
    t = jnp.maximum(
        jnp.dot(hg, w1_ref[...], preferred_element_type=jnp.float32)
        + b1_ref[...],
        0.0,
    )
    o_ref[...] = (
        jnp.dot(t, w2_ref[...], preferred_element_type=jnp.float32) + b2_ref[...]
    )


def _tc_pool(h, batch2, sb2, ns2, gi2, W1, b1, W2, b2):
    return pl.pallas_call(
        _pool_body, out_shape=jax.ShapeDtypeStruct((G, 1), jnp.float32)
    )(h, batch2, sb2, ns2, gi2, W1, b1, W2, b2)


def kernel(x, edge_index, edge_attr, batch, subgraph_batch, num_subgraphs,
           subgraph_id_batch, W_enc, b_enc, W_edge, b_edge, W_root, W_agg,
           b_gnn, gamma, beta, W1, b1, W2, b2):
    L = W_edge.shape[0]
    ei_r = edge_index.reshape(2, NW, NCHUNK, CW)

    h = _tc_encode(x, W_enc, b_enc.reshape(1, D))
    pre_p = _make_sc_pre()(ei_r, edge_attr)
    for l in range(L):
        P = _make_sc_spmm()(ei_r, h)
        h = _tc_layer(
            h, P, pre_p, W_edge[l], b_edge[l].reshape(1, D), W_root[l],
            W_agg[l], b_gnn[l].reshape(1, D), gamma[l].reshape(1, D),
            beta[l].reshape(1, D),
        )
    return _tc_pool(
        h, batch.reshape(N, 1), subgraph_batch.reshape(N, 1),
        num_subgraphs.reshape(G, 1), subgraph_id_batch.reshape(S, 1),
        W1, b1.reshape(1, 2 * D), W2, b2.reshape(1, 1),
    )


# speedup vs baseline: 6.9787x; 1.4110x over previous
import functools

import jax
import jax.numpy as jnp
from jax import lax
from jax.experimental import pallas as pl
from jax.experimental.pallas import tpu as pltpu
from jax.experimental.pallas import tpu_sc as plsc

N = 10000
E = 320000
D = 128
DE = 16
G = 64
S = 512
EPS = 1e-5

NC = 2
NS = 16
NW = NC * NS
EPW = E // NW
CW = 80
NCHUNK = EPW // CW
RPT = 624
REXT_START = NS * RPT
REXT = N - REXT_START
RZB = 104

@functools.cache
def _make_sc_spmm():
    mesh = plsc.VectorSubcoreMesh(
        core_axis_name="c", subcore_axis_name="s", num_cores=NC, num_subcores=NS
    )
    return functools.partial(
        pl.kernel,
        out_type=jax.ShapeDtypeStruct((NC, N, D), jnp.float32),
        mesh=mesh,
        scratch_types=[
            pltpu.VMEM((EPW,), jnp.int32),
            pltpu.VMEM((NCHUNK, CW), jnp.int32),
            pltpu.VMEM((CW, D), jnp.float32),
            pltpu.VMEM((CW, D), jnp.float32),
            pltpu.VMEM_SHARED((N, D), jnp.float32),
            pltpu.SemaphoreType.DMA,
        ],
    )(_sc_spmm_body)


def _sc_spmm_body(ei_flat, ei_hbm, h_hbm, out_hbm, src_v, dst_v, r0, r1, acc, sem):
    cid = lax.axis_index("c")
    sid = lax.axis_index("s")
    wid = cid * NS + sid
    pltpu.sync_copy(ei_flat.at[0, wid], src_v)
    pltpu.sync_copy(ei_hbm.at[1, wid], dst_v)

    @pl.loop(0, CW)
    def _zero_fill(i):
        for j in range(D // 16):
            r0[i, pl.ds(j * 16, 16)] = jnp.zeros((16,), jnp.float32)

    rs = pl.multiple_of(sid * RPT, 8)
    for m in range(RPT // CW):
        pltpu.sync_copy(r0, acc.at[pl.ds(rs + m * CW, CW)])
    pltpu.sync_copy(
        r0.at[pl.ds(0, RPT % CW)],
        acc.at[pl.ds(rs + (RPT // CW) * CW, RPT % CW)],
    )

    @pl.when(sid == NS - 1)
    def _zero_tail():
        pltpu.sync_copy(r0.at[pl.ds(0, REXT)], acc.at[pl.ds(REXT_START, REXT)])

    plsc.subcore_barrier()

    def _sidx(k):
        return src_v.at[pl.ds(pl.multiple_of(k * CW, 8), CW)]

    pltpu.async_copy(h_hbm.at[_sidx(0)], r0, sem)

    @pl.loop(0, (NCHUNK - 1) // 2)
    def _chunk(i):
        a = 2 * i
        pltpu.make_async_copy(h_hbm.at[_sidx(0)], r0, sem).wait()
        pltpu.async_copy(h_hbm.at[_sidx(a + 1)], r1, sem)
        pltpu.sync_copy(r0, acc.at[dst_v.at[a]], add=True)
        pltpu.async_copy(h_hbm.at[_sidx(a + 2)], r0, sem)
        pltpu.make_async_copy(h_hbm.at[_sidx(0)], r1, sem).wait()
        pltpu.sync_copy(r1, acc.at[dst_v.at[a + 1]], add=True)

    pltpu.make_async_copy(h_hbm.at[_sidx(0)], r0, sem).wait()
    pltpu.sync_copy(r0, acc.at[dst_v.at[NCHUNK - 1]], add=True)

    plsc.subcore_barrier()
    pltpu.sync_copy(acc.at[pl.ds(rs, RPT)], out_hbm.at[cid, pl.ds(rs, RPT)])

    @pl.when(sid == NS - 1)
    def _write_tail():
        pltpu.sync_copy(
            acc.at[pl.ds(REXT_START, REXT)],
            out_hbm.at[cid, pl.ds(REXT_START, REXT)],
        )


@functools.cache
def _make_sc_pre():
    mesh = plsc.VectorSubcoreMesh(
        core_axis_name="c", subcore_axis_name="s", num_cores=NC, num_subcores=NS
    )
    return functools.partial(
        pl.kernel,
        out_type=jax.ShapeDtypeStruct((NC, N, D), jnp.float32),
        mesh=mesh,
        scratch_types=[
            pltpu.VMEM((NCHUNK, CW), jnp.int32),
            pltpu.VMEM((CW, D), jnp.float32),
            pltpu.VMEM((CW, DE), jnp.float32),
            pltpu.VMEM_SHARED((N, D), jnp.float32),
            pltpu.SemaphoreType.DMA,
        ],
    )(_sc_pre_body)


def _sc_pre_body(ei_hbm, ea_hbm, out_hbm, dst_v, ud_v, ea_v, acc, sem):
    cid = lax.axis_index("c")
    sid = lax.axis_index("s")
    wid = cid * NS + sid
    pltpu.sync_copy(ei_hbm.at[1, wid], dst_v)

    @pl.loop(0, CW)
    def _zero_fill(i):
        for j in range(D // 16):
            ud_v[i, pl.ds(j * 16, 16)] = jnp.zeros((16,), jnp.float32)

    rs = pl.multiple_of(sid * RPT, 8)
    for m in range(RPT // CW):
        pltpu.sync_copy(ud_v, acc.at[pl.ds(rs + m * CW, CW)])
    pltpu.sync_copy(
        ud_v.at[pl.ds(0, RPT % CW)],
        acc.at[pl.ds(rs + (RPT // CW) * CW, RPT % CW)],
    )

    @pl.when(sid == NS - 1)
    def _zero_tail():
        pltpu.sync_copy(ud_v.at[pl.ds(0, REXT)], acc.at[pl.ds(REXT_START, REXT)])

    plsc.subcore_barrier()

    @pl.loop(0, CW)
    def _ones_fill(i):
        ud_v[i, pl.ds(DE, 16)] = jnp.ones((16,), jnp.float32)

    ebase = pl.multiple_of(wid * EPW, 8)

    @pl.loop(0, NCHUNK)
    def _chunk(k):
        pltpu.async_copy(
            ea_hbm.at[pl.ds(pl.multiple_of(ebase + k * CW, 8), CW)], ea_v, sem
        ).wait()

        @pl.loop(0, CW)
        def _fill(i):
            ud_v[i, pl.ds(0, DE)] = ea_v[i, :]

        pltpu.sync_copy(ud_v, acc.at[dst_v.at[k]], add=True)

    plsc.subcore_barrier()
    pltpu.sync_copy(acc.at[pl.ds(rs, RPT)], out_hbm.at[cid, pl.ds(rs, RPT)])

    @pl.when(sid == NS - 1)
    def _write_tail():
        pltpu.sync_copy(
            acc.at[pl.ds(REXT_START, REXT)], out_hbm.at[cid, pl.ds(REXT_START, REXT)]
        )


def _enc_body(x_ref, w_ref, b_ref, o_ref):
    o_ref[...] = (
        jnp.dot(x_ref[...], w_ref[...], preferred_element_type=jnp.float32)
        + b_ref[...]
    )


def _tc_encode(x, W_enc, b_enc):
    return pl.pallas_call(
        _enc_body, out_shape=jax.ShapeDtypeStruct((N, D), jnp.float32)
    )(x, W_enc, b_enc)


def _layer_body(h_ref, p_ref, pre_ref, we_ref, be_ref, wr_ref, wa_ref,
                bg_ref, ga_ref, bt_ref, o_ref):
    pre = pre_ref[0] + pre_ref[1]
    ea = pre[:, 0:DE]
    deg = pre[:, DE:DE + 1]
    agg = (
        p_ref[0]
        + p_ref[1]
        + jnp.dot(ea, we_ref[...], preferred_element_type=jnp.float32)
        + deg * be_ref[...]
    )
    z = (
        jnp.dot(h_ref[...], wr_ref[...], preferred_element_type=jnp.float32)
        + jnp.dot(agg, wa_ref[...], preferred_element_type=jnp.float32)
        + bg_ref[...]
    )
    mu = jnp.mean(z, axis=0, keepdims=True)
    zc = z - mu
    var = jnp.mean(zc * zc, axis=0, keepdims=True)
    zn = zc * lax.rsqrt(var + EPS) * ga_ref[...] + bt_ref[...]
    o_ref[...] = jnp.maximum(zn, 0.0)


def _tc_layer(h, P, pre_p, We, be, Wr, Wa, bg, ga, bt):
    return pl.pallas_call(
        _layer_body, out_shape=jax.ShapeDtypeStruct((N, D), jnp.float32)
    )(h, P, pre_p, We, be, Wr, Wa, bg, ga, bt)


def _pool_body(h_ref, b_ref, sb_ref, ns_ref, gi_ref, w1_ref, b1_ref, w2_ref,
               b2_ref, o_ref):
    ii = lax.broadcasted_iota(jnp.int32, (G, G), 0)
    jj = lax.broadcasted_iota(jnp.int32, (G, G), 1)
    tri = (jj < ii).astype(jnp.float32)
    offs_g = jnp.dot(tri, ns_ref[...].astype(jnp.float32),
                     preferred_element_type=jnp.float32)
    b_oh = (b_ref[...] == lax.broadcasted_iota(jnp.int32, (N, G), 1)).astype(
        jnp.float32
    )
    offs_n = jnp.dot(b_oh, offs_g, preferred_element_type=jnp.float32)
    sid = sb_ref[...].astype(jnp.float32) + offs_n
    oh = (
        sid == lax.broadcasted_iota(jnp.int32, (N, S), 1).astype(jnp.float32)
    ).astype(jnp.float32)
    dn = (((0,), (0,)), ((), ()))
    sums = lax.dot_general(oh, h_ref[...], dn,
                           preferred_element_type=jnp.float32)
    ones_n = jnp.zeros((N, 1), jnp.float32) + 1.0
    cnts = lax.dot_general(oh, ones_n, dn,
                           preferred_element_type=jnp.float32)
    hs = sums / jnp.maximum(cnts, 1.0)
    oh2 = (gi_ref[...] == lax.broadcasted_iota(jnp.int32, (S, G), 1)).astype(
        jnp.float32
    )
    gsum = lax.dot_general(oh2, hs, dn, preferred_element_type=jnp.float32)
    ones_s = jnp.zeros((S, 1), jnp.float32) + 1.0
    gcnt = lax.dot_general(oh2, ones_s, dn, preferred_element_type=jnp.float32)
    hg = gsum / jnp.maximum(gcnt, 1.0)
    t = jnp.maximum(
        jnp.dot(hg, w1_ref[...], preferred_element_type=jnp.float32)
        + b1_ref[...],
        0.0,
    )
    o_ref[...] = (
        jnp.dot(t, w2_ref[...], preferred_element_type=jnp.float32) + b2_ref[...]
    )


def _tc_pool(h, batch2, sb2, ns2, gi2, W1, b1, W2, b2):
    return pl.pallas_call(
        _pool_body, out_shape=jax.ShapeDtypeStruct((G, 1), jnp.float32)
    )(h, batch2, sb2, ns2, gi2, W1, b1, W2, b2)


def kernel(x, edge_index, edge_attr, batch, subgraph_batch, num_subgraphs,
           subgraph_id_batch, W_enc, b_enc, W_edge, b_edge, W_root, W_agg,
           b_gnn, gamma, beta, W1, b1, W2, b2):
    L = W_edge.shape[0]
    ei_r = edge_index.reshape(2, NW, NCHUNK, CW)
    ei_flat = edge_index.reshape(2, NW, EPW)

    h = _tc_encode(x, W_enc, b_enc.reshape(1, D))
    pre_p = _make_sc_pre()(ei_r, edge_attr)
    for l in range(L):
        P = _make_sc_spmm()(ei_flat, ei_r, h)
        h = _tc_layer(
            h, P, pre_p, W_edge[l], b_edge[l].reshape(1, D), W_root[l],
            W_agg[l], b_gnn[l].reshape(1, D), gamma[l].reshape(1, D),
            beta[l].reshape(1, D),
        )
    return _tc_pool(
        h, batch.reshape(N, 1), subgraph_batch.reshape(N, 1),
        num_subgraphs.reshape(G, 1), subgraph_id_batch.reshape(S, 1),
        W1, b1.reshape(1, 2 * D), W2, b2.reshape(1, 1),
    )
